# baseline (speedup 1.0000x reference)
import functools

import jax
import jax.numpy as jnp
from jax import lax
from jax.experimental import pallas as pl
from jax.experimental.pallas import tpu as pltpu
from jax.experimental.pallas import tpu_sc as plsc

EMBED_DIM = 64
SEQ_LEN = 200
BATCH = 1024
NUM_CORES = 2
NUM_SUBCORES = 16
NUM_WORKERS = NUM_CORES * NUM_SUBCORES
CHUNK = 128
TC_PER_L = BATCH // CHUNK
NBUF = 4
LEAD = 2
NWBUF = 2
LANES = 16
LTILES = SEQ_LEN // 8
LWIN = 7


def _sc_body(n_chunks, x_hbm, table2_hbm, pe_hbm, out_hbm,
             idx_v, pe_v, rows_v, obuf_v, gsems, wsems):
    wid = lax.axis_index("s") * NUM_CORES + lax.axis_index("c")
    ci0 = wid * n_chunks
    l0 = ci0 // TC_PER_L
    tl_base = lax.min(l0 // 8, LTILES - 2)

    pltpu.sync_copy(x_hbm.at[pl.ds(tl_base, 2)], idx_v)
    pltpu.sync_copy(pe_hbm.at[pl.ds(l0, LWIN)], pe_v)

    lane_iota = lax.iota(jnp.int32, LANES)

    def coords(j):
        ci = ci0 + j
        l = ci // TC_PER_L
        return l, lax.rem(ci, TC_PER_L), l // 8 - tl_base, lax.rem(l, 8)

    def chunk_idx(j):
        _, tc, rr, rl = coords(j)
        return idx_v.at[rr, tc, rl]

    def start_gather(j, b):
        pltpu.async_copy(
            table2_hbm.at[chunk_idx(j)], rows_v.at[b], gsems.at[b])

    def wait_gather(j, b):
        pltpu.make_async_copy(
            table2_hbm.at[chunk_idx(j)], rows_v.at[b], gsems.at[b]).wait()

    def start_write(j, w):
        l, tc, _, _ = coords(j)
        pltpu.async_copy(
            obuf_v.at[w],
            out_hbm.at[pl.ds(l, 1), pl.ds(0, EMBED_DIM // 8), pl.ds(tc, 1)],
            wsems.at[w])

    def wait_write(w):
        pltpu.make_async_copy(
            obuf_v.at[w],
            out_hbm.at[pl.ds(0, 1), pl.ds(0, EMBED_DIM // 8), pl.ds(0, 1)],
            wsems.at[w]).wait()

    def compute(j, b, w):
        l, _, _, _ = coords(j)
        rl_l = l - l0
        pe_q = [pe_v[rl_l, pl.ds(q * LANES, LANES)]
                for q in range(EMBED_DIM // LANES)]

        def batch_step(bb, carry):
            sl = pl.ds(bb * LANES, LANES)
            rowv = lane_iota + bb * LANES
            for d in range(EMBED_DIM):
                col = jnp.full((LANES,), d, jnp.int32)
                v = plsc.load_gather(rows_v.at[b], [rowv, col])
                pe_sc = pe_q[d // LANES][d % LANES]
                obuf_v[w, 0, d // 8, 0, d % 8, sl] = v + pe_sc
            return carry

        lax.fori_loop(0, CHUNK // LANES, batch_step, 0)

    def step(j, b, w, first_writes):
        jn = j + LEAD
        bn = (b + LEAD) % NBUF
        start_gather(jn, bn)
        if first_writes:
            wait_write(w)
        else:
            @pl.when(j >= NWBUF)
            def _():
                wait_write(w)
        wait_gather(j, b)
        compute(j, b, w)
        start_write(j, w)

    for j in range(LEAD):
        start_gather(j, j % NBUF)

    def outer(j4, carry):
        for b in range(NBUF):
            j = j4 * NBUF + b
            w = b % NWBUF
            step(j, b, w, first_writes=False)
        return carry

    lax.fori_loop(0, (n_chunks - LEAD) // NBUF, outer, 0)

    for j in range(n_chunks - LEAD, n_chunks):
        b = j % NBUF
        w = j % NWBUF
        wait_write(w)
        wait_gather(j, b)
        compute(j, b, w)
        start_write(j, w)

    for w in range(NWBUF):
        wait_write(w)


def kernel(x, token_table, pe_table):
    B, L = x.shape
    total = B * L
    n_chunks = total // (NUM_WORKERS * CHUNK)
    assert n_chunks * NUM_WORKERS * CHUNK == total
    assert (n_chunks - LEAD) % NBUF == 0
    assert B % CHUNK == 0 and L == SEQ_LEN and L % 8 == 0
    assert (n_chunks - 1) // TC_PER_L + 1 <= LWIN

    xn = (x.astype(jnp.int32)
          .reshape(TC_PER_L, CHUNK, LTILES, 8)
          .transpose(2, 0, 3, 1))
    pe_s = pe_table[:SEQ_LEN]
    V, D = token_table.shape
    table2 = jnp.pad(token_table, ((0, 0), (0, 128 - D)))

    mesh = plsc.VectorSubcoreMesh(core_axis_name="c", subcore_axis_name="s")
    run = pl.kernel(
        functools.partial(_sc_body, n_chunks),
        out_type=jax.ShapeDtypeStruct(
            (SEQ_LEN, EMBED_DIM // 8, TC_PER_L, 8, CHUNK), jnp.float32),
        mesh=mesh,
        compiler_params=pltpu.CompilerParams(
            use_tc_tiling_on_sc=False, needs_layout_passes=False),
        scratch_types=[
            pltpu.VMEM((2, TC_PER_L, 8, CHUNK), jnp.int32),
            pltpu.VMEM((LWIN, EMBED_DIM), jnp.float32),
            pltpu.VMEM((NBUF, CHUNK, 128), jnp.float32),
            pltpu.VMEM((NWBUF, 1, EMBED_DIM // 8, 1, 8, CHUNK),
                       jnp.float32),
            pltpu.SemaphoreType.DMA((NBUF,)),
            pltpu.SemaphoreType.DMA((NWBUF,)),
        ],
    )
    out = run(xn, table2, pe_s)
    return out.transpose((2, 4, 0, 1, 3)).reshape(B, L, EMBED_DIM)


# speedup vs baseline: 1.2028x; 1.2028x over previous
import functools

import jax
import jax.numpy as jnp
from jax import lax
from jax.experimental import pallas as pl
from jax.experimental.pallas import tpu as pltpu
from jax.experimental.pallas import tpu_sc as plsc

EMBED_DIM = 64
SEQ_LEN = 200
BATCH = 1024
NUM_CORES = 2
NUM_SUBCORES = 16
NUM_WORKERS = NUM_CORES * NUM_SUBCORES
CHUNK = 128
TC_PER_L = BATCH // CHUNK
NBUF = 5
LEAD = 2
LANES = 16
LTILES = SEQ_LEN // 8


def _sc_body(n_chunks, x_hbm, table_hbm, pe_hbm, out_hbm,
             idx_v, pe_v, rows_v, gsems, wsems):
    wid = lax.axis_index("s") * NUM_CORES + lax.axis_index("c")
    ci0 = wid * n_chunks
    l0 = ci0 // TC_PER_L
    tl_base = lax.min(l0 // 8, LTILES - 2)

    pltpu.sync_copy(x_hbm.at[pl.ds(tl_base, 2)], idx_v)
    pltpu.sync_copy(pe_hbm.at[pl.ds(0, SEQ_LEN)], pe_v)

    def chunk_idx(j):
        ci = ci0 + j
        l = ci // TC_PER_L
        tc = lax.rem(ci, TC_PER_L)
        return idx_v.at[l // 8 - tl_base, tc, lax.rem(l, 8)]

    def start_gather(j, b):
        pltpu.async_copy(table_hbm.at[chunk_idx(j)], rows_v.at[b], gsems.at[b])

    def wait_gather(j, b):
        pltpu.make_async_copy(
            table_hbm.at[chunk_idx(j)], rows_v.at[b], gsems.at[b]).wait()

    def start_write(j, b):
        ci = ci0 + j
        l = ci // TC_PER_L
        tc = lax.rem(ci, TC_PER_L)
        pltpu.async_copy(
            rows_v.at[b],
            out_hbm.at[pl.ds(tc * CHUNK, CHUNK),
                       pl.ds(l * EMBED_DIM, EMBED_DIM)],
            wsems.at[b])

    def wait_write(b):
        pltpu.make_async_copy(
            rows_v.at[b],
            out_hbm.at[pl.ds(0, CHUNK), pl.ds(0, EMBED_DIM)],
            wsems.at[b]).wait()

    def compute(j, b):
        ci = ci0 + j
        l = ci // TC_PER_L
        pe_q = [pe_v[l, pl.ds(q * LANES, LANES)]
                for q in range(EMBED_DIM // LANES)]

        def add_row(i, carry2):
            for q in range(EMBED_DIM // LANES):
                sl = pl.ds(q * LANES, LANES)
                rows_v[b, i, sl] = rows_v[b, i, sl] + pe_q[q]
            return carry2

        lax.fori_loop(0, CHUNK, add_row, 0, unroll=4)

    for j in range(LEAD):
        start_gather(j, j % NBUF)

    def outer(j5, carry):
        for b in range(NBUF):
            j = j5 * NBUF + b
            bn = (b + LEAD) % NBUF

            @pl.when(j + LEAD < n_chunks)
            def _():
                @pl.when(j >= NBUF - LEAD)
                def _():
                    wait_write(bn)
                start_gather(j + LEAD, bn)

            wait_gather(j, b)
            compute(j, b)
            start_write(j, b)
        return carry

    lax.fori_loop(0, n_chunks // NBUF, outer, 0)

    for b in range(NBUF):
        wait_write(b)


def kernel(x, token_table, pe_table):
    B, L = x.shape
    total = B * L
    n_chunks = total // (NUM_WORKERS * CHUNK)
    assert n_chunks * NUM_WORKERS * CHUNK == total
    assert n_chunks % NBUF == 0 and NBUF > LEAD
    assert B % CHUNK == 0 and L == SEQ_LEN and L % 8 == 0

    xn = (x.astype(jnp.int32)
          .reshape(TC_PER_L, CHUNK, LTILES, 8)
          .transpose(2, 0, 3, 1))
    pe_s = pe_table[:SEQ_LEN]

    mesh = plsc.VectorSubcoreMesh(core_axis_name="c", subcore_axis_name="s")
    run = pl.kernel(
        functools.partial(_sc_body, n_chunks),
        out_type=jax.ShapeDtypeStruct((B, L * EMBED_DIM), jnp.float32),
        mesh=mesh,
        compiler_params=pltpu.CompilerParams(use_tc_tiling_on_sc=False),
        scratch_types=[
            pltpu.VMEM((2, TC_PER_L, 8, CHUNK), jnp.int32),
            pltpu.VMEM((SEQ_LEN, EMBED_DIM), jnp.float32),
            pltpu.VMEM((NBUF, CHUNK, EMBED_DIM), jnp.float32),
            pltpu.SemaphoreType.DMA((NBUF,)),
            pltpu.SemaphoreType.DMA((NBUF,)),
        ],
    )
    out = run(xn, token_table, pe_s)
    return out.reshape(B, L, EMBED_DIM)
